# baseline (speedup 1.0000x reference)
import functools

import jax
import jax.numpy as jnp
from jax import lax
from jax.experimental import pallas as pl
from jax.experimental.pallas import tpu as pltpu
from jax.experimental.pallas import tpu_sc as plsc

N_NODES = 10000
N_EDGES = 320000
D = 128

NC = 2
NS = 16
NW = NC * NS

CHUNK = 128
NBUF = 3
K0 = 162
K1 = 0
E_PAD = (K0 + K1) * NS * CHUNK
ACC_ROWS = 10112
RPT = ACC_ROWS // NS

MM_BLK = 1000


def _transform_body(x_ref, w_ref, b_ref, o_ref):
    o_ref[...] = (
        jnp.dot(x_ref[...], w_ref[...], preferred_element_type=jnp.float32)
        + b_ref[...]
    )


_transform = pl.pallas_call(
    _transform_body,
    grid=(N_NODES // MM_BLK,),
    in_specs=[
        pl.BlockSpec((MM_BLK, D), lambda i: (i, 0)),
        pl.BlockSpec((D, 4 * D), lambda i: (0, 0)),
        pl.BlockSpec((1, 4 * D), lambda i: (0, 0)),
    ],
    out_specs=pl.BlockSpec((MM_BLK, 4 * D), lambda i: (i, 0)),
    out_shape=jax.ShapeDtypeStruct((N_NODES, 4 * D), jnp.float32),
)


def _combine_body(t_ref, p_ref, o_ref):
    o_ref[...] = jnp.maximum(t_ref[...] + p_ref[0] + p_ref[1], 0.0)


_combine = pl.pallas_call(
    _combine_body,
    grid=(N_NODES // MM_BLK,),
    in_specs=[
        pl.BlockSpec((MM_BLK, D), lambda i: (i, 3)),
        pl.BlockSpec((NC, MM_BLK, D), lambda i: (0, i, 0)),
    ],
    out_specs=pl.BlockSpec((MM_BLK, D), lambda i: (i, 0)),
    out_shape=jax.ShapeDtypeStruct((N_NODES, D), jnp.float32),
)


_sc_mesh = plsc.VectorSubcoreMesh(core_axis_name="c", subcore_axis_name="s")


@functools.partial(
    pl.kernel,
    out_type=jax.ShapeDtypeStruct((NC, ACC_ROWS, D), jnp.float32),
    mesh=_sc_mesh,
    scratch_types=(
        [pltpu.VMEM((CHUNK,), jnp.int32) for _ in range(NBUF)]
        + [pltpu.VMEM((CHUNK,), jnp.int32) for _ in range(NBUF)]
        + [pltpu.VMEM((CHUNK, D), jnp.float32) for _ in range(NBUF)]
        + [pltpu.VMEM_SHARED((ACC_ROWS, D), jnp.float32)]
        + [pltpu.SemaphoreType.DMA for _ in range(2 * NBUF)]
    ),
)
def _edge_scatter(table_hbm, g_hbm, dst_hbm, zero_hbm, out_hbm,
                  g0, g1, g2, d0, d1, d2, r0, r1, r2, acc,
                  si0, si1, si2, sg0, sg1, sg2):
    cid = lax.axis_index("c")
    sid = lax.axis_index("s")
    k = jnp.where(cid == 0, K0, K1)
    base = jnp.where(cid == 0, sid * K0, NS * K0 + sid * K1) * CHUNK

    g_bufs = (g0, g1, g2)
    d_bufs = (d0, d1, d2)
    r_bufs = (r0, r1, r2)
    si = (si0, si1, si2)
    sg = (sg0, sg1, sg2)

    def fire_idx(b, i):
        off = base + i * CHUNK
        pltpu.async_copy(g_hbm.at[pl.ds(off, CHUNK)], g_bufs[b], si[b])
        pltpu.async_copy(dst_hbm.at[pl.ds(off, CHUNK)], d_bufs[b], si[b])

    def wait_idx(b, i):
        off = base + i * CHUNK
        pltpu.make_async_copy(g_hbm.at[pl.ds(off, CHUNK)], g_bufs[b],
                              si[b]).wait()
        pltpu.make_async_copy(dst_hbm.at[pl.ds(off, CHUNK)], d_bufs[b],
                              si[b]).wait()

    def fire_gather(b):
        pltpu.async_copy(table_hbm.at[g_bufs[b]], r_bufs[b], sg[b])

    def wait_gather(b):
        pltpu.make_async_copy(table_hbm.at[g_bufs[b]], r_bufs[b],
                              sg[b]).wait()

    acc_rows = pl.ds(sid * RPT, RPT)
    pltpu.sync_copy(zero_hbm.at[acc_rows], acc.at[acc_rows])

    @pl.when(k > 0)
    def _():
        fire_idx(0, 0)
        fire_idx(1, 1)
        fire_idx(2, 2)
        wait_idx(0, 0)
        fire_gather(0)
        wait_idx(1, 1)
        fire_gather(1)

    plsc.subcore_barrier()

    @pl.loop(0, K0, step=NBUF)
    def _(i0):
        @pl.when(i0 < k)
        def _():
            for db in range(NBUF):
                b = db
                i = i0 + db
                wait_gather(b)
                pltpu.sync_copy(r_bufs[b], acc.at[d_bufs[b]], add=True)

                @pl.when(i + NBUF < k)
                def _():
                    fire_idx(b, i + NBUF)

                @pl.when(i + 2 < k)
                def _():
                    b2 = (db + 2) % NBUF
                    wait_idx(b2, i + 2)
                    fire_gather(b2)

    plsc.subcore_barrier()

    pltpu.sync_copy(acc.at[acc_rows], out_hbm.at[cid, acc_rows])


def kernel(x, edge_index, edge_type, W0, W1, W2, Ws, bs):
    x = x.astype(jnp.float32)
    src = edge_index[0].astype(jnp.int32)
    dst = edge_index[1].astype(jnp.int32)
    et = edge_type.astype(jnp.int32)

    pad = E_PAD - N_EDGES
    g = jnp.pad(src * 4 + et, (0, pad))
    dst = jnp.pad(dst, (0, pad), constant_values=N_NODES)
    zero = jnp.zeros((ACC_ROWS, D), jnp.float32)

    w_cat = jnp.concatenate([W0, W1, W2, Ws], axis=0).T
    b_cat = jnp.zeros((1, 4 * D), jnp.float32).at[0, 3 * D:].set(bs)

    table = _transform(x, w_cat, b_cat)
    partials = _edge_scatter(table.reshape(4 * N_NODES, D), g, dst, zero)
    return _combine(table, partials)


# speedup vs baseline: 3.9264x; 3.9264x over previous
import functools

import jax
import jax.numpy as jnp
from jax import lax
from jax.experimental import pallas as pl
from jax.experimental.pallas import tpu as pltpu
from jax.experimental.pallas import tpu_sc as plsc

N_NODES = 10000
N_EDGES = 320000
D = 128

NC = 2
NS = 16
NW = NC * NS

CHUNK = 128
NBUF = 3
NCHUNK = 81
EPT = NCHUNK * CHUNK
E_PAD = EPT * NW
N_DUMMY = 112
ACC_ROWS = 10112
RPT = ACC_ROWS // NS

MM_BLK = 1000


def _transform_body(x_ref, w_ref, b_ref, o_ref):
    o_ref[...] = (
        jnp.dot(x_ref[...], w_ref[...], preferred_element_type=jnp.float32)
        + b_ref[...]
    )


_transform = pl.pallas_call(
    _transform_body,
    grid=(N_NODES // MM_BLK,),
    in_specs=[
        pl.BlockSpec((MM_BLK, D), lambda i: (i, 0)),
        pl.BlockSpec((D, 4 * D), lambda i: (0, 0)),
        pl.BlockSpec((1, 4 * D), lambda i: (0, 0)),
    ],
    out_specs=pl.BlockSpec((MM_BLK, 4 * D), lambda i: (i, 0)),
    out_shape=jax.ShapeDtypeStruct((N_NODES, 4 * D), jnp.float32),
)


def _combine_body(t_ref, p_ref, o_ref):
    o_ref[...] = jnp.maximum(t_ref[...] + p_ref[0] + p_ref[1], 0.0)


_combine = pl.pallas_call(
    _combine_body,
    grid=(N_NODES // MM_BLK,),
    in_specs=[
        pl.BlockSpec((MM_BLK, D), lambda i: (i, 3)),
        pl.BlockSpec((NC, MM_BLK, D), lambda i: (0, i, 0)),
    ],
    out_specs=pl.BlockSpec((MM_BLK, D), lambda i: (i, 0)),
    out_shape=jax.ShapeDtypeStruct((N_NODES, D), jnp.float32),
)


_sc_mesh = plsc.VectorSubcoreMesh(core_axis_name="c", subcore_axis_name="s")


@functools.partial(
    pl.kernel,
    out_type=jax.ShapeDtypeStruct((NC, ACC_ROWS, D), jnp.float32),
    mesh=_sc_mesh,
    scratch_types=(
        [pltpu.VMEM((CHUNK,), jnp.int32) for _ in range(NBUF)]
        + [pltpu.VMEM((CHUNK,), jnp.int32) for _ in range(NBUF)]
        + [pltpu.VMEM((CHUNK, D), jnp.float32) for _ in range(NBUF)]
        + [pltpu.VMEM_SHARED((ACC_ROWS, D), jnp.float32)]
        + [pltpu.SemaphoreType.DMA for _ in range(2 * NBUF)]
    ),
)
def _edge_scatter(table_hbm, g_hbm, dst_hbm, zero_hbm, out_hbm,
                  g0, g1, g2, d0, d1, d2, r0, r1, r2, acc,
                  si0, si1, si2, sg0, sg1, sg2):
    cid = lax.axis_index("c")
    sid = lax.axis_index("s")
    base = (cid * NS + sid) * EPT

    g_bufs = (g0, g1, g2)
    d_bufs = (d0, d1, d2)
    r_bufs = (r0, r1, r2)
    si = (si0, si1, si2)
    sg = (sg0, sg1, sg2)

    def fire_idx(b, i):
        off = base + i * CHUNK
        pltpu.async_copy(g_hbm.at[pl.ds(off, CHUNK)], g_bufs[b], si[b])
        pltpu.async_copy(dst_hbm.at[pl.ds(off, CHUNK)], d_bufs[b], si[b])

    def wait_idx(b, i):
        off = base + i * CHUNK
        pltpu.make_async_copy(g_hbm.at[pl.ds(off, CHUNK)], g_bufs[b],
                              si[b]).wait()
        pltpu.make_async_copy(dst_hbm.at[pl.ds(off, CHUNK)], d_bufs[b],
                              si[b]).wait()

    def fire_gather(b):
        pltpu.async_copy(table_hbm.at[g_bufs[b]], r_bufs[b], sg[b])

    def wait_gather(b):
        pltpu.make_async_copy(table_hbm.at[g_bufs[b]], r_bufs[b],
                              sg[b]).wait()

    acc_rows = pl.ds(sid * RPT, RPT)
    pltpu.sync_copy(zero_hbm.at[acc_rows], acc.at[acc_rows])

    fire_idx(0, 0)
    fire_idx(1, 1)
    fire_idx(2, 2)
    wait_idx(0, 0)
    fire_gather(0)
    wait_idx(1, 1)
    fire_gather(1)

    plsc.subcore_barrier()

    @pl.loop(0, NCHUNK, step=NBUF)
    def _(i0):
        for db in range(NBUF):
            b = db
            i = i0 + db
            wait_gather(b)
            pltpu.sync_copy(r_bufs[b], acc.at[d_bufs[b]], add=True)

            @pl.when(i + NBUF < NCHUNK)
            def _():
                fire_idx(b, i + NBUF)

            @pl.when(i + 2 < NCHUNK)
            def _():
                b2 = (db + 2) % NBUF
                wait_idx(b2, i + 2)
                fire_gather(b2)

    plsc.subcore_barrier()

    pltpu.sync_copy(acc.at[acc_rows], out_hbm.at[cid, acc_rows])


def kernel(x, edge_index, edge_type, W0, W1, W2, Ws, bs):
    x = x.astype(jnp.float32)
    src = edge_index[0].astype(jnp.int32)
    dst = edge_index[1].astype(jnp.int32)
    et = edge_type.astype(jnp.int32)

    pad = E_PAD - N_EDGES
    pad_pos = jnp.arange(pad, dtype=jnp.int32)
    g = jnp.concatenate([src * 4 + et, pad_pos % 4096])
    dst = jnp.concatenate([dst, N_NODES + pad_pos % N_DUMMY])
    zero = jnp.zeros((ACC_ROWS, D), jnp.float32)

    w_cat = jnp.concatenate([W0, W1, W2, Ws], axis=0).T
    b_cat = jnp.zeros((1, 4 * D), jnp.float32).at[0, 3 * D:].set(bs)

    table = _transform(x, w_cat, b_cat)
    partials = _edge_scatter(table.reshape(4 * N_NODES, D), g, dst, zero)
    return _combine(table, partials)


# speedup vs baseline: 4.0478x; 1.0309x over previous
import functools

import jax
import jax.numpy as jnp
from jax import lax
from jax.experimental import pallas as pl
from jax.experimental.pallas import tpu as pltpu
from jax.experimental.pallas import tpu_sc as plsc

N_NODES = 10000
N_EDGES = 320000
D = 128

NC = 2
NS = 16
NW = NC * NS

CHUNK = 128
NBUF = 3
NCHUNK = 81
EPT = NCHUNK * CHUNK
E_PAD = EPT * NW
N_DUMMY = 112
ACC_ROWS = 10112
RPT = ACC_ROWS // NS

MM_BLK = 1000


def _transform_body(x_ref, w_ref, b_ref, o_ref):
    o_ref[...] = (
        jnp.dot(x_ref[...], w_ref[...], preferred_element_type=jnp.float32)
        + b_ref[...]
    )


_transform = pl.pallas_call(
    _transform_body,
    grid=(N_NODES // MM_BLK, 4),
    in_specs=[
        pl.BlockSpec((MM_BLK, D), lambda i, j: (i, 0)),
        pl.BlockSpec((D, D), lambda i, j: (0, j)),
        pl.BlockSpec((1, D), lambda i, j: (0, j)),
    ],
    out_specs=pl.BlockSpec((MM_BLK, D),
                           lambda i, j: (j * (N_NODES // MM_BLK) + i, 0)),
    out_shape=jax.ShapeDtypeStruct((4 * N_NODES, D), jnp.float32),
)


def _combine_body(t_ref, p_ref, o_ref):
    o_ref[...] = jnp.maximum(t_ref[...] + p_ref[0] + p_ref[1], 0.0)


_combine = pl.pallas_call(
    _combine_body,
    grid=(N_NODES // MM_BLK,),
    in_specs=[
        pl.BlockSpec((MM_BLK, D),
                     lambda i: (3 * (N_NODES // MM_BLK) + i, 0)),
        pl.BlockSpec((NC, MM_BLK, D), lambda i: (0, i, 0)),
    ],
    out_specs=pl.BlockSpec((MM_BLK, D), lambda i: (i, 0)),
    out_shape=jax.ShapeDtypeStruct((N_NODES, D), jnp.float32),
)


_sc_mesh = plsc.VectorSubcoreMesh(core_axis_name="c", subcore_axis_name="s")


@functools.partial(
    pl.kernel,
    out_type=jax.ShapeDtypeStruct((NC, ACC_ROWS, D), jnp.float32),
    mesh=_sc_mesh,
    scratch_types=(
        [pltpu.VMEM((CHUNK,), jnp.int32) for _ in range(NBUF)]
        + [pltpu.VMEM((CHUNK,), jnp.int32) for _ in range(NBUF)]
        + [pltpu.VMEM((CHUNK, D), jnp.float32) for _ in range(NBUF)]
        + [pltpu.VMEM_SHARED((ACC_ROWS, D), jnp.float32)]
        + [pltpu.SemaphoreType.DMA for _ in range(2 * NBUF)]
    ),
)
def _edge_scatter(table_hbm, g_hbm, dst_hbm, zero_hbm, out_hbm,
                  g0, g1, g2, d0, d1, d2, r0, r1, r2, acc,
                  si0, si1, si2, sg0, sg1, sg2):
    cid = lax.axis_index("c")
    sid = lax.axis_index("s")
    base = (cid * NS + sid) * EPT

    g_bufs = (g0, g1, g2)
    d_bufs = (d0, d1, d2)
    r_bufs = (r0, r1, r2)
    si = (si0, si1, si2)
    sg = (sg0, sg1, sg2)

    def fire_idx(b, i):
        off = base + i * CHUNK
        pltpu.async_copy(g_hbm.at[pl.ds(off, CHUNK)], g_bufs[b], si[b])
        pltpu.async_copy(dst_hbm.at[pl.ds(off, CHUNK)], d_bufs[b], si[b])

    def wait_idx(b, i):
        off = base + i * CHUNK
        pltpu.make_async_copy(g_hbm.at[pl.ds(off, CHUNK)], g_bufs[b],
                              si[b]).wait()
        pltpu.make_async_copy(dst_hbm.at[pl.ds(off, CHUNK)], d_bufs[b],
                              si[b]).wait()

    def fire_gather(b):
        pltpu.async_copy(table_hbm.at[g_bufs[b]], r_bufs[b], sg[b])

    def wait_gather(b):
        pltpu.make_async_copy(table_hbm.at[g_bufs[b]], r_bufs[b],
                              sg[b]).wait()

    acc_rows = pl.ds(sid * RPT, RPT)
    pltpu.sync_copy(zero_hbm.at[acc_rows], acc.at[acc_rows])

    fire_idx(0, 0)
    fire_idx(1, 1)
    fire_idx(2, 2)
    wait_idx(0, 0)
    fire_gather(0)
    wait_idx(1, 1)
    fire_gather(1)

    plsc.subcore_barrier()

    @pl.loop(0, NCHUNK, step=NBUF)
    def _(i0):
        for db in range(NBUF):
            b = db
            i = i0 + db
            wait_gather(b)
            pltpu.sync_copy(r_bufs[b], acc.at[d_bufs[b]], add=True)

            @pl.when(i + NBUF < NCHUNK)
            def _():
                fire_idx(b, i + NBUF)

            @pl.when(i + 2 < NCHUNK)
            def _():
                b2 = (db + 2) % NBUF
                wait_idx(b2, i + 2)
                fire_gather(b2)

    plsc.subcore_barrier()

    pltpu.sync_copy(acc.at[acc_rows], out_hbm.at[cid, acc_rows])


def kernel(x, edge_index, edge_type, W0, W1, W2, Ws, bs):
    x = x.astype(jnp.float32)
    src = edge_index[0].astype(jnp.int32)
    dst = edge_index[1].astype(jnp.int32)
    et = edge_type.astype(jnp.int32)

    pad = E_PAD - N_EDGES
    pad_pos = jnp.arange(pad, dtype=jnp.int32)
    g = jnp.concatenate([et * N_NODES + src, pad_pos % 4096])
    dst = jnp.concatenate([dst, N_NODES + pad_pos % N_DUMMY])
    zero = jnp.zeros((ACC_ROWS, D), jnp.float32)

    w_cat = jnp.concatenate([W0, W1, W2, Ws], axis=0).T
    b_cat = jnp.zeros((1, 4 * D), jnp.float32).at[0, 3 * D:].set(bs)

    table = _transform(x, w_cat, b_cat)
    partials = _edge_scatter(table, g, dst, zero)
    return _combine(table, partials)
